# Initial kernel scaffold; baseline (speedup 1.0000x reference)
import jax
import jax.numpy as jnp
from jax.experimental import pallas as pl


def kernel(node_features, edge_index, edge_features, clique_features, node2clique_index, clique_edge_index, clique_edge_features, params):
    raise NotImplementedError("write your pallas kernel here")


# speedup vs baseline: 1.1256x; 1.1256x over previous
import functools

import jax
import jax.numpy as jnp
from jax.experimental import pallas as pl

_N_NODES = 10000
_N_EDGES = 160000
_N_CLIQUES = 2000
_N_N2C = 20000
_N_CEDGES = 16000
_D_NODE = 128
_MSG = 4
_D_EDGE = 16
_D_CLQ = 32
_D_CEDGE = 16
_AVG_DEG_LIN = 16.0
_AVG_DEG_LOG = 2.833213344056216

_EB = 2000
_EB_C = 2000
_NB = 2000

_f32 = jnp.float32


def _pna_edge_body(eattr_ref, xsrc_ref, w1_ref, b1_ref, w2r_ref, b2m_ref, out_ref):
    h = jnp.maximum(
        jnp.dot(eattr_ref[...], w1_ref[...], preferred_element_type=_f32)
        + b1_ref[...], 0.0)
    xs = xsrc_ref[...]
    cols = []
    for m in range(_MSG):
        t = jnp.dot(h, w2r_ref[m], preferred_element_type=_f32)
        cols.append(jnp.sum(t * xs, axis=1, keepdims=True))
    out_ref[...] = (jnp.concatenate(cols, axis=1)
                    + jnp.dot(xs, b2m_ref[...], preferred_element_type=_f32))


def _pna_edge(eattr, xsrc, w1, b1, w2r, b2m):
    grid = _N_EDGES // _EB
    return pl.pallas_call(
        _pna_edge_body,
        grid=(grid,),
        in_specs=[
            pl.BlockSpec((_EB, _D_EDGE), lambda i: (i, 0)),
            pl.BlockSpec((_EB, _D_NODE), lambda i: (i, 0)),
            pl.BlockSpec((_D_EDGE, _D_NODE), lambda i: (0, 0)),
            pl.BlockSpec((1, _D_NODE), lambda i: (0, 0)),
            pl.BlockSpec((_MSG, _D_NODE, _D_NODE), lambda i: (0, 0, 0)),
            pl.BlockSpec((_D_NODE, _MSG), lambda i: (0, 0)),
        ],
        out_specs=pl.BlockSpec((_EB, _MSG), lambda i: (i, 0)),
        out_shape=jax.ShapeDtypeStruct((_N_EDGES, _MSG), _f32),
    )(eattr, xsrc, w1, b1, w2r, b2m)


def _node_post_body(x_ref, f_ref, wt_ref, wb_ref, b_ref, out_ref):
    out_ref[...] = (
        jnp.dot(x_ref[...], wt_ref[...], preferred_element_type=_f32)
        + jnp.dot(f_ref[...], wb_ref[...], preferred_element_type=_f32)
        + b_ref[...])


def _node_post(x, feats, wt, wb, b):
    grid = _N_NODES // _NB
    return pl.pallas_call(
        _node_post_body,
        grid=(grid,),
        in_specs=[
            pl.BlockSpec((_NB, _D_NODE), lambda i: (i, 0)),
            pl.BlockSpec((_NB, 16 * _MSG), lambda i: (i, 0)),
            pl.BlockSpec((_D_NODE, _D_NODE), lambda i: (0, 0)),
            pl.BlockSpec((16 * _MSG, _D_NODE), lambda i: (0, 0)),
            pl.BlockSpec((1, _D_NODE), lambda i: (0, 0)),
        ],
        out_specs=pl.BlockSpec((_NB, _D_NODE), lambda i: (i, 0)),
        out_shape=jax.ShapeDtypeStruct((_N_NODES, _D_NODE), _f32),
    )(x, feats, wt, wb, b)


def _clique_pre_body(c_ref, s_ref, icnt_ref, w_ref, b_ref, out_ref):
    agg = s_ref[...] * icnt_ref[...]
    out_ref[...] = c_ref[...] + jnp.maximum(
        jnp.dot(agg, w_ref[...], preferred_element_type=_f32) + b_ref[...], 0.0)


def _clique_pre(c, s, icnt, w, b):
    return pl.pallas_call(
        _clique_pre_body,
        in_specs=[
            pl.BlockSpec((_N_CLIQUES, _D_CLQ), lambda: (0, 0)),
            pl.BlockSpec((_N_CLIQUES, _D_NODE), lambda: (0, 0)),
            pl.BlockSpec((_N_CLIQUES, 1), lambda: (0, 0)),
            pl.BlockSpec((_D_NODE, _D_CLQ), lambda: (0, 0)),
            pl.BlockSpec((1, _D_CLQ), lambda: (0, 0)),
        ],
        out_specs=pl.BlockSpec((_N_CLIQUES, _D_CLQ), lambda: (0, 0)),
        out_shape=jax.ShapeDtypeStruct((_N_CLIQUES, _D_CLQ), _f32),
    )(c, s, icnt, w, b)


def _nnconv_edge_body(eattr_ref, csrc_ref, w1_ref, b1_ref, w2r_ref, b2m_ref, out_ref):
    h = jnp.maximum(
        jnp.dot(eattr_ref[...], w1_ref[...], preferred_element_type=_f32)
        + b1_ref[...], 0.0)
    cs = csrc_ref[...]
    acc = jnp.dot(cs, b2m_ref[...], preferred_element_type=_f32)
    for k in range(_D_CLQ):
        acc = acc + h[:, k:k + 1] * jnp.dot(
            cs, w2r_ref[k], preferred_element_type=_f32)
    out_ref[...] = acc


def _nnconv_edge(eattr, csrc, w1, b1, w2r, b2m):
    grid = _N_CEDGES // _EB_C
    return pl.pallas_call(
        _nnconv_edge_body,
        grid=(grid,),
        in_specs=[
            pl.BlockSpec((_EB_C, _D_CEDGE), lambda i: (i, 0)),
            pl.BlockSpec((_EB_C, _D_CLQ), lambda i: (i, 0)),
            pl.BlockSpec((_D_CEDGE, _D_CLQ), lambda i: (0, 0)),
            pl.BlockSpec((1, _D_CLQ), lambda i: (0, 0)),
            pl.BlockSpec((_D_CLQ, _D_CLQ, _D_CLQ), lambda i: (0, 0, 0)),
            pl.BlockSpec((_D_CLQ, _D_CLQ), lambda i: (0, 0)),
        ],
        out_specs=pl.BlockSpec((_EB_C, _D_CLQ), lambda i: (i, 0)),
        out_shape=jax.ShapeDtypeStruct((_N_CEDGES, _D_CLQ), _f32),
    )(eattr, csrc, w1, b1, w2r, b2m)


def _clique_out_body(c_ref, aggr_ref, root_ref, bias_ref, out_ref):
    out_ref[...] = (
        jnp.dot(c_ref[...], root_ref[...], preferred_element_type=_f32)
        + aggr_ref[...] + bias_ref[...])


def _clique_out(c, aggr, root, bias):
    return pl.pallas_call(
        _clique_out_body,
        in_specs=[
            pl.BlockSpec((_N_CLIQUES, _D_CLQ), lambda: (0, 0)),
            pl.BlockSpec((_N_CLIQUES, _D_CLQ), lambda: (0, 0)),
            pl.BlockSpec((_D_CLQ, _D_CLQ), lambda: (0, 0)),
            pl.BlockSpec((1, _D_CLQ), lambda: (0, 0)),
        ],
        out_specs=pl.BlockSpec((_N_CLIQUES, _D_CLQ), lambda: (0, 0)),
        out_shape=jax.ShapeDtypeStruct((_N_CLIQUES, _D_CLQ), _f32),
    )(c, aggr, root, bias)


def _node_add_body(x_ref, cm_ref, w_ref, b_ref, has_ref, out_ref):
    out_ref[...] = (
        x_ref[...]
        + jnp.dot(cm_ref[...], w_ref[...], preferred_element_type=_f32)
        + has_ref[...] * b_ref[...])


def _node_add(x, cm, w, b, has):
    grid = _N_NODES // _NB
    return pl.pallas_call(
        _node_add_body,
        grid=(grid,),
        in_specs=[
            pl.BlockSpec((_NB, _D_NODE), lambda i: (i, 0)),
            pl.BlockSpec((_NB, _D_CLQ), lambda i: (i, 0)),
            pl.BlockSpec((_D_CLQ, _D_NODE), lambda i: (0, 0)),
            pl.BlockSpec((1, _D_NODE), lambda i: (0, 0)),
            pl.BlockSpec((_NB, 1), lambda i: (i, 0)),
        ],
        out_specs=pl.BlockSpec((_NB, _D_NODE), lambda i: (i, 0)),
        out_shape=jax.ShapeDtypeStruct((_N_NODES, _D_NODE), _f32),
    )(x, cm, w, b, has)


def kernel(node_features, edge_index, edge_features, clique_features,
           node2clique_index, clique_edge_index, clique_edge_features, params):
    x = node_features
    c = clique_features
    src, dst = edge_index[0], edge_index[1]
    csrc, cdst = clique_edge_index[0], clique_edge_index[1]
    nidx, cidx = node2clique_index[0], node2clique_index[1]

    deg = jax.ops.segment_sum(jnp.ones((_N_EDGES,), _f32), dst, _N_NODES)
    degc = jnp.maximum(deg, 1.0)[:, None]
    idegc = 1.0 / degc
    logd = jnp.log(deg + 1.0)
    amp = (logd / _AVG_DEG_LOG)[:, None]
    att = jnp.where(logd > 0.0, _AVG_DEG_LOG / jnp.where(logd > 0.0, logd, 1.0),
                    1.0)[:, None]
    lin = (deg / _AVG_DEG_LIN)[:, None]
    has_edge = (deg > 0.0)[:, None]

    cnt_c = jnp.maximum(
        jax.ops.segment_sum(jnp.ones((_N_N2C,), _f32), cidx, _N_CLIQUES), 1.0)
    icnt_c = (1.0 / cnt_c)[:, None]
    cnt_ce = jnp.maximum(
        jax.ops.segment_sum(jnp.ones((_N_CEDGES,), _f32), cdst, _N_CLIQUES), 1.0)
    icnt_ce = (1.0 / cnt_ce)[:, None]
    cnt_n = jax.ops.segment_sum(jnp.ones((_N_N2C,), _f32), nidx, _N_NODES)
    icnt_n = (1.0 / jnp.maximum(cnt_n, 1.0))[:, None]
    has_n = (cnt_n > 0.0).astype(_f32)[:, None]

    for P in params:
        w2r = jnp.stack([P["pna_w2"][:, m::_MSG] for m in range(_MSG)])
        b2m = P["pna_b2"].reshape(_D_NODE, _MSG)
        xsrc = x[src]
        msg = _pna_edge(edge_features, xsrc, P["pna_w1"],
                        P["pna_b1"][None, :], w2r, b2m)
        s1 = jax.ops.segment_sum(msg, dst, _N_NODES)
        s2 = jax.ops.segment_sum(msg * msg, dst, _N_NODES)
        mx = jax.ops.segment_max(msg, dst, _N_NODES)
        mn = jax.ops.segment_min(msg, dst, _N_NODES)
        mean = s1 * idegc
        meansq = s2 * idegc
        std = jnp.sqrt(jnp.maximum(meansq - mean * mean, 0.0) + 1e-5)
        mx = jnp.where(has_edge, mx, 0.0)
        mn = jnp.where(has_edge, mn, 0.0)
        agg = jnp.concatenate([mean, mx, mn, std], axis=1)
        feats = jnp.concatenate([agg, agg * amp, agg * att, agg * lin], axis=1)
        x = _node_post(x, feats, P["pna_post_w"][:_D_NODE],
                       P["pna_post_w"][_D_NODE:], P["pna_post_b"][None, :])

        s_c = jax.ops.segment_sum(x[nidx], cidx, _N_CLIQUES)
        c = _clique_pre(c, s_c, icnt_c, P["n2c_w"], P["n2c_b"][None, :])

        w2rc = P["nnc_w2"].reshape(_D_CLQ, _D_CLQ, _D_CLQ)
        b2mc = P["nnc_b2"].reshape(_D_CLQ, _D_CLQ)
        cmsg = _nnconv_edge(clique_edge_features, c[csrc], P["nnc_w1"],
                            P["nnc_b1"][None, :], w2rc, b2mc)
        aggr = jax.ops.segment_sum(cmsg, cdst, _N_CLIQUES) * icnt_ce
        c = _clique_out(c, aggr, P["nnc_root"], P["nnc_bias"][None, :])

        cm = jax.ops.segment_sum(c[cidx], nidx, _N_NODES) * icnt_n
        x = _node_add(x, cm, P["c2n_w"], P["c2n_b"][None, :], has_n)

    return (x, c)


# speedup vs baseline: 1.6823x; 1.4946x over previous
import functools

import jax
import jax.numpy as jnp
from jax import lax
from jax.experimental import pallas as pl
from jax.experimental.pallas import tpu as pltpu
from jax.experimental.pallas import tpu_sc as plsc

_N_NODES = 10000
_N_EDGES = 160000
_N_CLIQUES = 2000
_N_N2C = 20000
_N_CEDGES = 16000
_D_NODE = 128
_MSG = 4
_D_EDGE = 16
_D_CLQ = 32
_D_CEDGE = 16
_AVG_DEG_LIN = 16.0
_AVG_DEG_LOG = 2.833213344056216

_EB = 2000
_EB_C = 2000
_NB = 2000

_f32 = jnp.float32


def _pna_edge_body(eattr_ref, xsrc_ref, w1_ref, b1_ref, w2r_ref, b2m_ref, out_ref):
    h = jnp.maximum(
        jnp.dot(eattr_ref[...], w1_ref[...], preferred_element_type=_f32)
        + b1_ref[...], 0.0)
    xs = xsrc_ref[...]
    cols = []
    for m in range(_MSG):
        t = jnp.dot(h, w2r_ref[m], preferred_element_type=_f32)
        cols.append(jnp.sum(t * xs, axis=1, keepdims=True))
    msg = (jnp.concatenate(cols, axis=1)
           + jnp.dot(xs, b2m_ref[...], preferred_element_type=_f32))
    out_ref[...] = jnp.concatenate([msg, msg * msg], axis=1)


def _pna_edge(eattr, xsrc, w1, b1, w2r, b2m):
    grid = _N_EDGES // _EB
    return pl.pallas_call(
        _pna_edge_body,
        grid=(grid,),
        in_specs=[
            pl.BlockSpec((_EB, _D_EDGE), lambda i: (i, 0)),
            pl.BlockSpec((_EB, _D_NODE), lambda i: (i, 0)),
            pl.BlockSpec((_D_EDGE, _D_NODE), lambda i: (0, 0)),
            pl.BlockSpec((1, _D_NODE), lambda i: (0, 0)),
            pl.BlockSpec((_MSG, _D_NODE, _D_NODE), lambda i: (0, 0, 0)),
            pl.BlockSpec((_D_NODE, _MSG), lambda i: (0, 0)),
        ],
        out_specs=pl.BlockSpec((_EB, 2 * _MSG), lambda i: (i, 0)),
        out_shape=jax.ShapeDtypeStruct((_N_EDGES, 2 * _MSG), _f32),
    )(eattr, xsrc, w1, b1, w2r, b2m)


_SC_NC = 2
_SC_NS = 16
_SC_NW = _SC_NC * _SC_NS
_SC_CH = 128


@functools.lru_cache(maxsize=None)
def _make_sc_segsum(d, n_out_pad, k, identity):
    rows_w = k * _SC_CH
    rpt = n_out_pad // _SC_NS

    @functools.partial(
        pl.kernel,
        mesh=plsc.VectorSubcoreMesh(core_axis_name="c", subcore_axis_name="s"),
        compiler_params=pltpu.CompilerParams(use_tc_tiling_on_sc=False),
        out_type=jax.ShapeDtypeStruct((_SC_NC, n_out_pad, d), _f32),
        scratch_types=[
            pltpu.VMEM((k, _SC_CH), jnp.int32),
            pltpu.VMEM((k, _SC_CH), jnp.int32),
            pltpu.VMEM((rows_w, d), _f32),
            pltpu.VMEM_SHARED((n_out_pad, d), _f32),
            pltpu.SemaphoreType.DMA,
        ],
    )
    def fn(table_hbm, gidx_hbm, sidx_hbm, zeros_hbm, out_hbm,
           sidx_v, gidx_v, rows_v, acc_sh, sem):
        cid = lax.axis_index("c")
        sid = lax.axis_index("s")
        wid = cid * _SC_NS + sid
        zsl = pl.ds(sid * rpt, rpt)
        pltpu.sync_copy(zeros_hbm.at[zsl], acc_sh.at[zsl])
        pltpu.sync_copy(sidx_hbm.at[wid], sidx_v)
        if identity:
            pltpu.sync_copy(table_hbm.at[pl.ds(wid * rows_w, rows_w)], rows_v)
        else:
            pltpu.sync_copy(gidx_hbm.at[wid], gidx_v)

            def gbody(j, carry):
                pltpu.async_copy(table_hbm.at[gidx_v.at[j]],
                                 rows_v.at[pl.ds(j * _SC_CH, _SC_CH)],
                                 sem).wait()
                return carry

            lax.fori_loop(0, k, gbody, 0)
        plsc.subcore_barrier()

        def sbody(j, carry):
            pltpu.sync_copy(rows_v.at[pl.ds(j * _SC_CH, _SC_CH)],
                            acc_sh.at[sidx_v.at[j]], add=True)
            return carry

        lax.fori_loop(0, k, sbody, 0)
        plsc.subcore_barrier()
        pltpu.sync_copy(acc_sh.at[zsl], out_hbm.at[cid, zsl])

    return fn


def _round_up(v, m):
    return (v + m - 1) // m * m


def _sc_segsum(table, gidx, sidx, n_out):
    d = table.shape[1]
    p = sidx.shape[0]
    k = _round_up(p, _SC_NW * _SC_CH) // (_SC_NW * _SC_CH)
    p_pad = _SC_NW * _SC_CH * k
    n_out_pad = _round_up(n_out + 1, _SC_NS * 8)
    sidx_p = jnp.concatenate(
        [sidx, jnp.full((p_pad - p,), n_out, jnp.int32)]).reshape(
            _SC_NW, k, _SC_CH)
    identity = gidx is None
    if identity:
        tbl = jnp.concatenate(
            [table, jnp.zeros((p_pad - p, d), _f32)], axis=0)
        gidx_p = sidx_p
    else:
        tbl = table
        gidx_p = jnp.concatenate(
            [gidx, jnp.zeros((p_pad - p,), jnp.int32)]).reshape(
                _SC_NW, k, _SC_CH)
    zeros = jnp.zeros((n_out_pad, d), _f32)
    fn = _make_sc_segsum(d, n_out_pad, k, identity)
    out = fn(tbl, gidx_p, sidx_p, zeros)
    return out[0, :n_out] + out[1, :n_out]


def _node_post_body(x_ref, f_ref, wt_ref, wb_ref, b_ref, out_ref):
    out_ref[...] = (
        jnp.dot(x_ref[...], wt_ref[...], preferred_element_type=_f32)
        + jnp.dot(f_ref[...], wb_ref[...], preferred_element_type=_f32)
        + b_ref[...])


def _node_post(x, feats, wt, wb, b):
    grid = _N_NODES // _NB
    return pl.pallas_call(
        _node_post_body,
        grid=(grid,),
        in_specs=[
            pl.BlockSpec((_NB, _D_NODE), lambda i: (i, 0)),
            pl.BlockSpec((_NB, 16 * _MSG), lambda i: (i, 0)),
            pl.BlockSpec((_D_NODE, _D_NODE), lambda i: (0, 0)),
            pl.BlockSpec((16 * _MSG, _D_NODE), lambda i: (0, 0)),
            pl.BlockSpec((1, _D_NODE), lambda i: (0, 0)),
        ],
        out_specs=pl.BlockSpec((_NB, _D_NODE), lambda i: (i, 0)),
        out_shape=jax.ShapeDtypeStruct((_N_NODES, _D_NODE), _f32),
    )(x, feats, wt, wb, b)


def _clique_pre_body(c_ref, s_ref, icnt_ref, w_ref, b_ref, out_ref):
    agg = s_ref[...] * icnt_ref[...]
    out_ref[...] = c_ref[...] + jnp.maximum(
        jnp.dot(agg, w_ref[...], preferred_element_type=_f32) + b_ref[...], 0.0)


def _clique_pre(c, s, icnt, w, b):
    return pl.pallas_call(
        _clique_pre_body,
        in_specs=[
            pl.BlockSpec((_N_CLIQUES, _D_CLQ), lambda: (0, 0)),
            pl.BlockSpec((_N_CLIQUES, _D_NODE), lambda: (0, 0)),
            pl.BlockSpec((_N_CLIQUES, 1), lambda: (0, 0)),
            pl.BlockSpec((_D_NODE, _D_CLQ), lambda: (0, 0)),
            pl.BlockSpec((1, _D_CLQ), lambda: (0, 0)),
        ],
        out_specs=pl.BlockSpec((_N_CLIQUES, _D_CLQ), lambda: (0, 0)),
        out_shape=jax.ShapeDtypeStruct((_N_CLIQUES, _D_CLQ), _f32),
    )(c, s, icnt, w, b)


def _nnconv_edge_body(eattr_ref, csrc_ref, w1_ref, b1_ref, w2r_ref, b2m_ref, out_ref):
    h = jnp.maximum(
        jnp.dot(eattr_ref[...], w1_ref[...], preferred_element_type=_f32)
        + b1_ref[...], 0.0)
    cs = csrc_ref[...]
    acc = jnp.dot(cs, b2m_ref[...], preferred_element_type=_f32)
    for k in range(_D_CLQ):
        acc = acc + h[:, k:k + 1] * jnp.dot(
            cs, w2r_ref[k], preferred_element_type=_f32)
    out_ref[...] = acc


def _nnconv_edge(eattr, csrc, w1, b1, w2r, b2m):
    grid = _N_CEDGES // _EB_C
    return pl.pallas_call(
        _nnconv_edge_body,
        grid=(grid,),
        in_specs=[
            pl.BlockSpec((_EB_C, _D_CEDGE), lambda i: (i, 0)),
            pl.BlockSpec((_EB_C, _D_CLQ), lambda i: (i, 0)),
            pl.BlockSpec((_D_CEDGE, _D_CLQ), lambda i: (0, 0)),
            pl.BlockSpec((1, _D_CLQ), lambda i: (0, 0)),
            pl.BlockSpec((_D_CLQ, _D_CLQ, _D_CLQ), lambda i: (0, 0, 0)),
            pl.BlockSpec((_D_CLQ, _D_CLQ), lambda i: (0, 0)),
        ],
        out_specs=pl.BlockSpec((_EB_C, _D_CLQ), lambda i: (i, 0)),
        out_shape=jax.ShapeDtypeStruct((_N_CEDGES, _D_CLQ), _f32),
    )(eattr, csrc, w1, b1, w2r, b2m)


def _clique_out_body(c_ref, aggr_ref, root_ref, bias_ref, out_ref):
    out_ref[...] = (
        jnp.dot(c_ref[...], root_ref[...], preferred_element_type=_f32)
        + aggr_ref[...] + bias_ref[...])


def _clique_out(c, aggr, root, bias):
    return pl.pallas_call(
        _clique_out_body,
        in_specs=[
            pl.BlockSpec((_N_CLIQUES, _D_CLQ), lambda: (0, 0)),
            pl.BlockSpec((_N_CLIQUES, _D_CLQ), lambda: (0, 0)),
            pl.BlockSpec((_D_CLQ, _D_CLQ), lambda: (0, 0)),
            pl.BlockSpec((1, _D_CLQ), lambda: (0, 0)),
        ],
        out_specs=pl.BlockSpec((_N_CLIQUES, _D_CLQ), lambda: (0, 0)),
        out_shape=jax.ShapeDtypeStruct((_N_CLIQUES, _D_CLQ), _f32),
    )(c, aggr, root, bias)


def _node_add_body(x_ref, cm_ref, w_ref, b_ref, has_ref, out_ref):
    out_ref[...] = (
        x_ref[...]
        + jnp.dot(cm_ref[...], w_ref[...], preferred_element_type=_f32)
        + has_ref[...] * b_ref[...])


def _node_add(x, cm, w, b, has):
    grid = _N_NODES // _NB
    return pl.pallas_call(
        _node_add_body,
        grid=(grid,),
        in_specs=[
            pl.BlockSpec((_NB, _D_NODE), lambda i: (i, 0)),
            pl.BlockSpec((_NB, _D_CLQ), lambda i: (i, 0)),
            pl.BlockSpec((_D_CLQ, _D_NODE), lambda i: (0, 0)),
            pl.BlockSpec((1, _D_NODE), lambda i: (0, 0)),
            pl.BlockSpec((_NB, 1), lambda i: (i, 0)),
        ],
        out_specs=pl.BlockSpec((_NB, _D_NODE), lambda i: (i, 0)),
        out_shape=jax.ShapeDtypeStruct((_N_NODES, _D_NODE), _f32),
    )(x, cm, w, b, has)


def kernel(node_features, edge_index, edge_features, clique_features,
           node2clique_index, clique_edge_index, clique_edge_features, params):
    x = node_features
    c = clique_features
    src, dst = edge_index[0], edge_index[1]
    csrc, cdst = clique_edge_index[0], clique_edge_index[1]
    nidx, cidx = node2clique_index[0], node2clique_index[1]

    deg = jax.ops.segment_sum(jnp.ones((_N_EDGES,), _f32), dst, _N_NODES)
    degc = jnp.maximum(deg, 1.0)[:, None]
    idegc = 1.0 / degc
    logd = jnp.log(deg + 1.0)
    amp = (logd / _AVG_DEG_LOG)[:, None]
    att = jnp.where(logd > 0.0, _AVG_DEG_LOG / jnp.where(logd > 0.0, logd, 1.0),
                    1.0)[:, None]
    lin = (deg / _AVG_DEG_LIN)[:, None]
    has_edge = (deg > 0.0)[:, None]

    cnt_c = jnp.maximum(
        jax.ops.segment_sum(jnp.ones((_N_N2C,), _f32), cidx, _N_CLIQUES), 1.0)
    icnt_c = (1.0 / cnt_c)[:, None]
    cnt_ce = jnp.maximum(
        jax.ops.segment_sum(jnp.ones((_N_CEDGES,), _f32), cdst, _N_CLIQUES), 1.0)
    icnt_ce = (1.0 / cnt_ce)[:, None]
    cnt_n = jax.ops.segment_sum(jnp.ones((_N_N2C,), _f32), nidx, _N_NODES)
    icnt_n = (1.0 / jnp.maximum(cnt_n, 1.0))[:, None]
    has_n = (cnt_n > 0.0).astype(_f32)[:, None]

    for P in params:
        w2r = jnp.stack([P["pna_w2"][:, m::_MSG] for m in range(_MSG)])
        b2m = P["pna_b2"].reshape(_D_NODE, _MSG)
        xsrc = x[src]
        msg8 = _pna_edge(edge_features, xsrc, P["pna_w1"],
                         P["pna_b1"][None, :], w2r, b2m)
        msg = msg8[:, :_MSG]
        s8 = _sc_segsum(msg8, None, dst, _N_NODES)
        s1, s2 = s8[:, :_MSG], s8[:, _MSG:]
        mx = jax.ops.segment_max(msg, dst, _N_NODES)
        mn = jax.ops.segment_min(msg, dst, _N_NODES)
        mean = s1 * idegc
        meansq = s2 * idegc
        std = jnp.sqrt(jnp.maximum(meansq - mean * mean, 0.0) + 1e-5)
        mx = jnp.where(has_edge, mx, 0.0)
        mn = jnp.where(has_edge, mn, 0.0)
        agg = jnp.concatenate([mean, mx, mn, std], axis=1)
        feats = jnp.concatenate([agg, agg * amp, agg * att, agg * lin], axis=1)
        x = _node_post(x, feats, P["pna_post_w"][:_D_NODE],
                       P["pna_post_w"][_D_NODE:], P["pna_post_b"][None, :])

        s_c = _sc_segsum(x, nidx, cidx, _N_CLIQUES)
        c = _clique_pre(c, s_c, icnt_c, P["n2c_w"], P["n2c_b"][None, :])

        w2rc = P["nnc_w2"].reshape(_D_CLQ, _D_CLQ, _D_CLQ)
        b2mc = P["nnc_b2"].reshape(_D_CLQ, _D_CLQ)
        cmsg = _nnconv_edge(clique_edge_features, c[csrc], P["nnc_w1"],
                            P["nnc_b1"][None, :], w2rc, b2mc)
        aggr = _sc_segsum(cmsg, None, cdst, _N_CLIQUES) * icnt_ce
        c = _clique_out(c, aggr, P["nnc_root"], P["nnc_bias"][None, :])

        cm = _sc_segsum(c, cidx, nidx, _N_NODES) * icnt_n
        x = _node_add(x, cm, P["c2n_w"], P["c2n_b"][None, :], has_n)

    return (x, c)


# speedup vs baseline: 2.4988x; 1.4853x over previous
import functools

import jax
import jax.numpy as jnp
from jax import lax
from jax.experimental import pallas as pl
from jax.experimental.pallas import tpu as pltpu
from jax.experimental.pallas import tpu_sc as plsc

_N_NODES = 10000
_N_EDGES = 160000
_N_CLIQUES = 2000
_N_N2C = 20000
_N_CEDGES = 16000
_D_NODE = 128
_MSG = 4
_D_EDGE = 16
_D_CLQ = 32
_D_CEDGE = 16
_AVG_DEG_LIN = 16.0
_AVG_DEG_LOG = 2.833213344056216

_EB = 2000
_EB_C = 2000
_NB = 2000

_f32 = jnp.float32


def _pna_edge_body(eattr_ref, xsrc_ref, w1_ref, b1_ref, w2r_ref, b2m_ref, out_ref):
    h = jnp.maximum(
        jnp.dot(eattr_ref[...], w1_ref[...], preferred_element_type=_f32)
        + b1_ref[...], 0.0)
    xs = xsrc_ref[...]
    cols = []
    for m in range(_MSG):
        t = jnp.dot(h, w2r_ref[m], preferred_element_type=_f32)
        cols.append(jnp.sum(t * xs, axis=1, keepdims=True))
    msg = (jnp.concatenate(cols, axis=1)
           + jnp.dot(xs, b2m_ref[...], preferred_element_type=_f32))
    out_ref[...] = jnp.concatenate([msg, msg * msg], axis=1)


def _pna_edge(eattr, xsrc, w1, b1, w2r, b2m):
    grid = _N_EDGES // _EB
    return pl.pallas_call(
        _pna_edge_body,
        grid=(grid,),
        in_specs=[
            pl.BlockSpec((_EB, _D_EDGE), lambda i: (i, 0)),
            pl.BlockSpec((_EB, _D_NODE), lambda i: (i, 0)),
            pl.BlockSpec((_D_EDGE, _D_NODE), lambda i: (0, 0)),
            pl.BlockSpec((1, _D_NODE), lambda i: (0, 0)),
            pl.BlockSpec((_MSG, _D_NODE, _D_NODE), lambda i: (0, 0, 0)),
            pl.BlockSpec((_D_NODE, _MSG), lambda i: (0, 0)),
        ],
        out_specs=pl.BlockSpec((_EB, 2 * _MSG), lambda i: (i, 0)),
        out_shape=jax.ShapeDtypeStruct((_N_EDGES, 2 * _MSG), _f32),
    )(eattr, xsrc, w1, b1, w2r, b2m)


_SC_NC = 2
_SC_NS = 16
_SC_NW = _SC_NC * _SC_NS
_SC_CH = 128


@functools.lru_cache(maxsize=None)
def _make_sc_segsum(d, n_out_pad, k, identity):
    rows_w = k * _SC_CH
    rpt = n_out_pad // _SC_NS

    @functools.partial(
        pl.kernel,
        mesh=plsc.VectorSubcoreMesh(core_axis_name="c", subcore_axis_name="s"),
        compiler_params=pltpu.CompilerParams(use_tc_tiling_on_sc=False),
        out_type=jax.ShapeDtypeStruct((_SC_NC, n_out_pad, d), _f32),
        scratch_types=[
            pltpu.VMEM((k, _SC_CH), jnp.int32),
            pltpu.VMEM((k, _SC_CH), jnp.int32),
            pltpu.VMEM((rows_w, d), _f32),
            pltpu.VMEM_SHARED((n_out_pad, d), _f32),
            pltpu.SemaphoreType.DMA,
        ],
    )
    def fn(table_hbm, gidx_hbm, sidx_hbm, zeros_hbm, out_hbm,
           sidx_v, gidx_v, rows_v, acc_sh, sem):
        cid = lax.axis_index("c")
        sid = lax.axis_index("s")
        wid = cid * _SC_NS + sid
        zsl = pl.ds(sid * rpt, rpt)
        pltpu.sync_copy(zeros_hbm.at[zsl], acc_sh.at[zsl])
        pltpu.sync_copy(sidx_hbm.at[wid], sidx_v)
        if identity:
            pltpu.sync_copy(table_hbm.at[pl.ds(wid * rows_w, rows_w)], rows_v)
        else:
            pltpu.sync_copy(gidx_hbm.at[wid], gidx_v)

            def gbody(j, carry):
                pltpu.async_copy(table_hbm.at[gidx_v.at[j]],
                                 rows_v.at[pl.ds(j * _SC_CH, _SC_CH)],
                                 sem).wait()
                return carry

            lax.fori_loop(0, k, gbody, 0)
        plsc.subcore_barrier()

        def sbody(j, carry):
            pltpu.sync_copy(rows_v.at[pl.ds(j * _SC_CH, _SC_CH)],
                            acc_sh.at[sidx_v.at[j]], add=True)
            return carry

        lax.fori_loop(0, k, sbody, 0)
        plsc.subcore_barrier()
        pltpu.sync_copy(acc_sh.at[zsl], out_hbm.at[cid, zsl])

    return fn


def _round_up(v, m):
    return (v + m - 1) // m * m


_PNA_K = 40
_PNA_EPAD = _SC_NW * _PNA_K * _SC_CH
_PNA_NPAD = _round_up(_N_NODES + 1, _SC_NS * 8)
_PNA_ACC = _PNA_NPAD * _MSG
_PNA_SUMPAD = _round_up(_N_NODES + 1, _SC_NS * 8 * 8)
_FMAX = 3.0e38


def _take16(v, idx):
    dn = lax.GatherDimensionNumbers(offset_dims=(), collapsed_slice_dims=(0,),
                                    start_index_map=(0,))
    return lax.gather(v, idx[:, None], dn, (1,),
                      mode=lax.GatherScatterMode.PROMISE_IN_BOUNDS)


@functools.lru_cache(maxsize=None)
def _make_sc_pna_reduce():
    rows_w = _PNA_K * _SC_CH
    rpt = _PNA_SUMPAD // _SC_NS

    @functools.partial(
        pl.kernel,
        mesh=plsc.VectorSubcoreMesh(core_axis_name="c", subcore_axis_name="s"),
        compiler_params=pltpu.CompilerParams(use_tc_tiling_on_sc=False,
                                             needs_layout_passes=False),
        out_type=(
            jax.ShapeDtypeStruct((_SC_NC, _PNA_SUMPAD, 8), _f32),
            jax.ShapeDtypeStruct((_SC_NW, _PNA_ACC), _f32),
            jax.ShapeDtypeStruct((_SC_NW, _PNA_ACC), _f32),
        ),
        scratch_types=[
            pltpu.VMEM((_PNA_K, _SC_CH), jnp.int32),
            pltpu.VMEM((_SC_CH, 8), _f32),
            pltpu.VMEM((_PNA_ACC,), _f32),
            pltpu.VMEM((_PNA_ACC,), _f32),
            pltpu.VMEM_SHARED((_PNA_SUMPAD, 8), _f32),
            pltpu.SemaphoreType.DMA,
        ],
    )
    def fn(msg_hbm, sidx_hbm, zeros_hbm, neg_hbm, pos_hbm,
           sums_hbm, mx_hbm, mn_hbm,
           sidx_v, chunk_v, amax_v, amin_v, acc_sh, sem):
        cid = lax.axis_index("c")
        sid = lax.axis_index("s")
        wid = cid * _SC_NS + sid
        zsl = pl.ds(sid * rpt, rpt)
        pltpu.sync_copy(zeros_hbm.at[zsl], acc_sh.at[zsl])
        pltpu.sync_copy(neg_hbm, amax_v)
        pltpu.sync_copy(pos_hbm, amin_v)
        pltpu.sync_copy(sidx_hbm.at[wid], sidx_v)
        plsc.subcore_barrier()

        lane = lax.iota(jnp.int32, 16)
        shifts = (1, 2, 4, 8)
        up_idx = [jnp.maximum(lane - sh, 0) for sh in shifts]
        ge_m = [lane >= sh for sh in shifts]
        dn_idx = jnp.minimum(lane + 1, 15)
        is_last = lane == 15

        def body(j, carry):
            pltpu.sync_copy(
                msg_hbm.at[pl.ds(wid * rows_w + j * _SC_CH, _SC_CH)],
                chunk_v)
            pltpu.sync_copy(chunk_v, acc_sh.at[sidx_v.at[j]], add=True)
            jfull = jnp.full((16,), j, jnp.int32)
            for s in range(8):
                key = plsc.load_gather(sidx_v, [jfull, s * 16 + lane])
                skey, perm = plsc.sort_key_val(key, lane)
                rowi = s * 16 + perm
                sk_up = [_take16(skey, ui) for ui in up_idx]
                eq = [(skey == sk_up[t]) & ge_m[t] for t in range(4)]
                lead = (skey != _take16(skey, dn_idx)) | is_last
                for m in range(_MSG):
                    mcol = jnp.full((16,), m, jnp.int32)
                    v = plsc.load_gather(chunk_v, [rowi, mcol])
                    vmx = v
                    vmn = v
                    for t in range(4):
                        tx = _take16(vmx, up_idx[t])
                        vmx = jnp.where(eq[t], jnp.maximum(vmx, tx), vmx)
                        tn = _take16(vmn, up_idx[t])
                        vmn = jnp.where(eq[t], jnp.minimum(vmn, tn), vmn)
                    flat = skey * _MSG + mcol
                    cur = plsc.load_gather(amax_v, [flat], mask=lead)
                    plsc.store_scatter(amax_v, [flat],
                                       jnp.maximum(cur, vmx), mask=lead)
                    cur2 = plsc.load_gather(amin_v, [flat], mask=lead)
                    plsc.store_scatter(amin_v, [flat],
                                       jnp.minimum(cur2, vmn), mask=lead)
            return carry

        lax.fori_loop(0, _PNA_K, body, 0)
        plsc.subcore_barrier()
        pltpu.sync_copy(acc_sh.at[zsl], sums_hbm.at[cid, zsl])
        pltpu.sync_copy(amax_v, mx_hbm.at[wid])
        pltpu.sync_copy(amin_v, mn_hbm.at[wid])

    return fn


def _sc_pna_reduce(msg8, dst):
    pad = _PNA_EPAD - _N_EDGES
    msg_p = jnp.concatenate([msg8, jnp.zeros((pad, 8), _f32)], axis=0)
    sidx_p = jnp.concatenate(
        [dst, jnp.full((pad,), _N_NODES, jnp.int32)]).reshape(
            _SC_NW, _PNA_K, _SC_CH)
    zeros = jnp.zeros((_PNA_SUMPAD, 8), _f32)
    neg = jnp.full((_PNA_ACC,), -_FMAX, _f32)
    pos = jnp.full((_PNA_ACC,), _FMAX, _f32)
    sums, mx32, mn32 = _make_sc_pna_reduce()(
        msg_p, sidx_p, zeros, neg, pos)
    s8 = (sums[0] + sums[1])[:_N_NODES]
    mx = jnp.max(mx32.reshape(_SC_NW, _PNA_NPAD, _MSG), axis=0)[:_N_NODES]
    mn = jnp.min(mn32.reshape(_SC_NW, _PNA_NPAD, _MSG), axis=0)[:_N_NODES]
    return s8, mx, mn


def _sc_segsum(table, gidx, sidx, n_out):
    d = table.shape[1]
    p = sidx.shape[0]
    k = _round_up(p, _SC_NW * _SC_CH) // (_SC_NW * _SC_CH)
    p_pad = _SC_NW * _SC_CH * k
    n_out_pad = _round_up(n_out + 1, _SC_NS * 8)
    sidx_p = jnp.concatenate(
        [sidx, jnp.full((p_pad - p,), n_out, jnp.int32)]).reshape(
            _SC_NW, k, _SC_CH)
    identity = gidx is None
    if identity:
        tbl = jnp.concatenate(
            [table, jnp.zeros((p_pad - p, d), _f32)], axis=0)
        gidx_p = sidx_p
    else:
        tbl = table
        gidx_p = jnp.concatenate(
            [gidx, jnp.zeros((p_pad - p,), jnp.int32)]).reshape(
                _SC_NW, k, _SC_CH)
    zeros = jnp.zeros((n_out_pad, d), _f32)
    fn = _make_sc_segsum(d, n_out_pad, k, identity)
    out = fn(tbl, gidx_p, sidx_p, zeros)
    return out[0, :n_out] + out[1, :n_out]


def _node_post_body(x_ref, f_ref, wt_ref, wb_ref, b_ref, out_ref):
    out_ref[...] = (
        jnp.dot(x_ref[...], wt_ref[...], preferred_element_type=_f32)
        + jnp.dot(f_ref[...], wb_ref[...], preferred_element_type=_f32)
        + b_ref[...])


def _node_post(x, feats, wt, wb, b):
    grid = _N_NODES // _NB
    return pl.pallas_call(
        _node_post_body,
        grid=(grid,),
        in_specs=[
            pl.BlockSpec((_NB, _D_NODE), lambda i: (i, 0)),
            pl.BlockSpec((_NB, 16 * _MSG), lambda i: (i, 0)),
            pl.BlockSpec((_D_NODE, _D_NODE), lambda i: (0, 0)),
            pl.BlockSpec((16 * _MSG, _D_NODE), lambda i: (0, 0)),
            pl.BlockSpec((1, _D_NODE), lambda i: (0, 0)),
        ],
        out_specs=pl.BlockSpec((_NB, _D_NODE), lambda i: (i, 0)),
        out_shape=jax.ShapeDtypeStruct((_N_NODES, _D_NODE), _f32),
    )(x, feats, wt, wb, b)


def _clique_pre_body(c_ref, s_ref, icnt_ref, w_ref, b_ref, out_ref):
    agg = s_ref[...] * icnt_ref[...]
    out_ref[...] = c_ref[...] + jnp.maximum(
        jnp.dot(agg, w_ref[...], preferred_element_type=_f32) + b_ref[...], 0.0)


def _clique_pre(c, s, icnt, w, b):
    return pl.pallas_call(
        _clique_pre_body,
        in_specs=[
            pl.BlockSpec((_N_CLIQUES, _D_CLQ), lambda: (0, 0)),
            pl.BlockSpec((_N_CLIQUES, _D_NODE), lambda: (0, 0)),
            pl.BlockSpec((_N_CLIQUES, 1), lambda: (0, 0)),
            pl.BlockSpec((_D_NODE, _D_CLQ), lambda: (0, 0)),
            pl.BlockSpec((1, _D_CLQ), lambda: (0, 0)),
        ],
        out_specs=pl.BlockSpec((_N_CLIQUES, _D_CLQ), lambda: (0, 0)),
        out_shape=jax.ShapeDtypeStruct((_N_CLIQUES, _D_CLQ), _f32),
    )(c, s, icnt, w, b)


def _nnconv_edge_body(eattr_ref, csrc_ref, w1_ref, b1_ref, w2r_ref, b2m_ref, out_ref):
    h = jnp.maximum(
        jnp.dot(eattr_ref[...], w1_ref[...], preferred_element_type=_f32)
        + b1_ref[...], 0.0)
    cs = csrc_ref[...]
    acc = jnp.dot(cs, b2m_ref[...], preferred_element_type=_f32)
    for k in range(_D_CLQ):
        acc = acc + h[:, k:k + 1] * jnp.dot(
            cs, w2r_ref[k], preferred_element_type=_f32)
    out_ref[...] = acc


def _nnconv_edge(eattr, csrc, w1, b1, w2r, b2m):
    grid = _N_CEDGES // _EB_C
    return pl.pallas_call(
        _nnconv_edge_body,
        grid=(grid,),
        in_specs=[
            pl.BlockSpec((_EB_C, _D_CEDGE), lambda i: (i, 0)),
            pl.BlockSpec((_EB_C, _D_CLQ), lambda i: (i, 0)),
            pl.BlockSpec((_D_CEDGE, _D_CLQ), lambda i: (0, 0)),
            pl.BlockSpec((1, _D_CLQ), lambda i: (0, 0)),
            pl.BlockSpec((_D_CLQ, _D_CLQ, _D_CLQ), lambda i: (0, 0, 0)),
            pl.BlockSpec((_D_CLQ, _D_CLQ), lambda i: (0, 0)),
        ],
        out_specs=pl.BlockSpec((_EB_C, _D_CLQ), lambda i: (i, 0)),
        out_shape=jax.ShapeDtypeStruct((_N_CEDGES, _D_CLQ), _f32),
    )(eattr, csrc, w1, b1, w2r, b2m)


def _clique_out_body(c_ref, aggr_ref, root_ref, bias_ref, out_ref):
    out_ref[...] = (
        jnp.dot(c_ref[...], root_ref[...], preferred_element_type=_f32)
        + aggr_ref[...] + bias_ref[...])


def _clique_out(c, aggr, root, bias):
    return pl.pallas_call(
        _clique_out_body,
        in_specs=[
            pl.BlockSpec((_N_CLIQUES, _D_CLQ), lambda: (0, 0)),
            pl.BlockSpec((_N_CLIQUES, _D_CLQ), lambda: (0, 0)),
            pl.BlockSpec((_D_CLQ, _D_CLQ), lambda: (0, 0)),
            pl.BlockSpec((1, _D_CLQ), lambda: (0, 0)),
        ],
        out_specs=pl.BlockSpec((_N_CLIQUES, _D_CLQ), lambda: (0, 0)),
        out_shape=jax.ShapeDtypeStruct((_N_CLIQUES, _D_CLQ), _f32),
    )(c, aggr, root, bias)


def _node_add_body(x_ref, cm_ref, w_ref, b_ref, has_ref, out_ref):
    out_ref[...] = (
        x_ref[...]
        + jnp.dot(cm_ref[...], w_ref[...], preferred_element_type=_f32)
        + has_ref[...] * b_ref[...])


def _node_add(x, cm, w, b, has):
    grid = _N_NODES // _NB
    return pl.pallas_call(
        _node_add_body,
        grid=(grid,),
        in_specs=[
            pl.BlockSpec((_NB, _D_NODE), lambda i: (i, 0)),
            pl.BlockSpec((_NB, _D_CLQ), lambda i: (i, 0)),
            pl.BlockSpec((_D_CLQ, _D_NODE), lambda i: (0, 0)),
            pl.BlockSpec((1, _D_NODE), lambda i: (0, 0)),
            pl.BlockSpec((_NB, 1), lambda i: (i, 0)),
        ],
        out_specs=pl.BlockSpec((_NB, _D_NODE), lambda i: (i, 0)),
        out_shape=jax.ShapeDtypeStruct((_N_NODES, _D_NODE), _f32),
    )(x, cm, w, b, has)


def kernel(node_features, edge_index, edge_features, clique_features,
           node2clique_index, clique_edge_index, clique_edge_features, params):
    x = node_features
    c = clique_features
    src, dst = edge_index[0], edge_index[1]
    csrc, cdst = clique_edge_index[0], clique_edge_index[1]
    nidx, cidx = node2clique_index[0], node2clique_index[1]

    deg = jax.ops.segment_sum(jnp.ones((_N_EDGES,), _f32), dst, _N_NODES)
    degc = jnp.maximum(deg, 1.0)[:, None]
    idegc = 1.0 / degc
    logd = jnp.log(deg + 1.0)
    amp = (logd / _AVG_DEG_LOG)[:, None]
    att = jnp.where(logd > 0.0, _AVG_DEG_LOG / jnp.where(logd > 0.0, logd, 1.0),
                    1.0)[:, None]
    lin = (deg / _AVG_DEG_LIN)[:, None]
    has_edge = (deg > 0.0)[:, None]

    cnt_c = jnp.maximum(
        jax.ops.segment_sum(jnp.ones((_N_N2C,), _f32), cidx, _N_CLIQUES), 1.0)
    icnt_c = (1.0 / cnt_c)[:, None]
    cnt_ce = jnp.maximum(
        jax.ops.segment_sum(jnp.ones((_N_CEDGES,), _f32), cdst, _N_CLIQUES), 1.0)
    icnt_ce = (1.0 / cnt_ce)[:, None]
    cnt_n = jax.ops.segment_sum(jnp.ones((_N_N2C,), _f32), nidx, _N_NODES)
    icnt_n = (1.0 / jnp.maximum(cnt_n, 1.0))[:, None]
    has_n = (cnt_n > 0.0).astype(_f32)[:, None]

    for P in params:
        w2r = jnp.stack([P["pna_w2"][:, m::_MSG] for m in range(_MSG)])
        b2m = P["pna_b2"].reshape(_D_NODE, _MSG)
        xsrc = x[src]
        msg8 = _pna_edge(edge_features, xsrc, P["pna_w1"],
                         P["pna_b1"][None, :], w2r, b2m)
        s8, mx, mn = _sc_pna_reduce(msg8, dst)
        s1, s2 = s8[:, :_MSG], s8[:, _MSG:]
        mean = s1 * idegc
        meansq = s2 * idegc
        std = jnp.sqrt(jnp.maximum(meansq - mean * mean, 0.0) + 1e-5)
        mx = jnp.where(has_edge, mx, 0.0)
        mn = jnp.where(has_edge, mn, 0.0)
        agg = jnp.concatenate([mean, mx, mn, std], axis=1)
        feats = jnp.concatenate([agg, agg * amp, agg * att, agg * lin], axis=1)
        x = _node_post(x, feats, P["pna_post_w"][:_D_NODE],
                       P["pna_post_w"][_D_NODE:], P["pna_post_b"][None, :])

        s_c = _sc_segsum(x, nidx, cidx, _N_CLIQUES)
        c = _clique_pre(c, s_c, icnt_c, P["n2c_w"], P["n2c_b"][None, :])

        w2rc = P["nnc_w2"].reshape(_D_CLQ, _D_CLQ, _D_CLQ)
        b2mc = P["nnc_b2"].reshape(_D_CLQ, _D_CLQ)
        cmsg = _nnconv_edge(clique_edge_features, c[csrc], P["nnc_w1"],
                            P["nnc_b1"][None, :], w2rc, b2mc)
        aggr = _sc_segsum(cmsg, None, cdst, _N_CLIQUES) * icnt_ce
        c = _clique_out(c, aggr, P["nnc_root"], P["nnc_bias"][None, :])

        cm = _sc_segsum(c, cidx, nidx, _N_NODES) * icnt_n
        x = _node_add(x, cm, P["c2n_w"], P["c2n_b"][None, :], has_n)

    return (x, c)
